# Initial kernel scaffold; baseline (speedup 1.0000x reference)
import jax
import jax.numpy as jnp
from jax.experimental import pallas as pl


def kernel(predicted_locs, predicted_scores, gt_locs, gt_labels):
    raise NotImplementedError("write your pallas kernel here")


# speedup vs baseline: 1.2230x; 1.2230x over previous
import jax
import jax.numpy as jnp
from jax.experimental import pallas as pl
from jax.experimental.pallas import tpu as pltpu

_C = 81
_NEG_POS_RATIO = 3.0


def _stage1_body(scores_ref, labels_ref, plocs_ref, glocs_ref,
                 ce_ref, npos_ref, locsum_ref, cepos_ref):
    x = scores_ref[...]
    lab = labels_ref[...]
    p = x.shape[0]

    mx = jnp.max(x, axis=1, keepdims=True)
    e = jnp.exp(x - mx)
    se = jnp.sum(e, axis=1, keepdims=True)
    lse = jnp.log(se) + mx
    cls = jax.lax.broadcasted_iota(jnp.int32, x.shape, 1)
    xlab = jnp.sum(jnp.where(cls == lab, x, 0.0), axis=1, keepdims=True)
    ce = lse - xlab
    pos = lab > 0

    d = plocs_ref[...] - glocs_ref[...]
    ad = jnp.abs(d)
    sl1 = jnp.where(ad < 1.0, 0.5 * d * d, ad - 0.5)
    loc_l1 = jnp.sum(sl1, axis=1, keepdims=True)

    ce_ref[...] = jnp.where(pos, 0.0, ce).reshape(1, p)
    npos_ref[...] = jnp.sum(jnp.where(pos, 1.0, 0.0)).reshape(1, 1)
    locsum_ref[...] = jnp.sum(jnp.where(pos, loc_l1, 0.0)).reshape(1, 1)
    cepos_ref[...] = jnp.sum(jnp.where(pos, ce, 0.0)).reshape(1, 1)


def _stage2_body(ce_ref, npos_ref, locsum_ref, cepos_ref, out_ref):
    ce = jnp.maximum(ce_ref[...], 0.0)
    b, p = ce.shape
    npos = npos_ref[...][:, :, 0]
    negcnt = p - npos
    k = jnp.minimum(_NEG_POS_RATIO * npos, negcnt)

    loc_sum = jnp.sum(locsum_ref[...])
    ce_pos = jnp.sum(cepos_ref[...])
    total_pos = jnp.maximum(jnp.sum(npos), 1.0)

    rowsum = jnp.sum(ce, axis=1, keepdims=True)
    fast = jnp.all((k >= negcnt) | (k <= 0.0))

    @pl.when(fast)
    def _():
        conf = jnp.sum(jnp.where(k > 0.0, rowsum, 0.0))
        out_ref[...] = jnp.reshape((loc_sum + ce_pos + conf) / total_pos, (1, 1))

    @pl.when(jnp.logical_not(fast))
    def _():
        u = jax.lax.bitcast_convert_type(ce, jnp.int32)
        lo0 = jnp.zeros((b, 1), jnp.int32)
        hi0 = jnp.full((b, 1), jnp.int32(0x7F800000))

        def step(_, carry):
            lo, hi = carry
            mid = lo + jax.lax.shift_right_logical(hi - lo, 1)
            cnt = jnp.sum(jnp.where(u >= mid, 1.0, 0.0), axis=1, keepdims=True)
            go = cnt >= k
            return jnp.where(go, mid, lo), jnp.where(go, hi, mid)

        lo, _ = jax.lax.fori_loop(0, 31, step, (lo0, hi0))
        t = jax.lax.bitcast_convert_type(lo, jnp.float32)
        gt = ce > t
        cnt_gt = jnp.sum(jnp.where(gt, 1.0, 0.0), axis=1, keepdims=True)
        sum_gt = jnp.sum(jnp.where(gt, ce, 0.0), axis=1, keepdims=True)
        topk = sum_gt + (k - cnt_gt) * t
        conf = jnp.sum(jnp.where(k > 0.0, topk, 0.0))
        out_ref[...] = jnp.reshape((loc_sum + ce_pos + conf) / total_pos, (1, 1))


def kernel(predicted_locs, predicted_scores, gt_locs, gt_labels):
    B, P, C = predicted_scores.shape
    labels3 = gt_labels.astype(jnp.int32).reshape(B, P, 1)

    ce_neg, npos, locsum, cepos = pl.pallas_call(
        _stage1_body,
        grid=(B,),
        in_specs=[
            pl.BlockSpec((None, P, C), lambda b: (b, 0, 0)),
            pl.BlockSpec((None, P, 1), lambda b: (b, 0, 0)),
            pl.BlockSpec((None, P, 4), lambda b: (b, 0, 0)),
            pl.BlockSpec((None, P, 4), lambda b: (b, 0, 0)),
        ],
        out_specs=[
            pl.BlockSpec((None, 1, P), lambda b: (b, 0, 0)),
            pl.BlockSpec((None, 1, 1), lambda b: (b, 0, 0)),
            pl.BlockSpec((None, 1, 1), lambda b: (b, 0, 0)),
            pl.BlockSpec((None, 1, 1), lambda b: (b, 0, 0)),
        ],
        out_shape=[
            jax.ShapeDtypeStruct((B, 1, P), jnp.float32),
            jax.ShapeDtypeStruct((B, 1, 1), jnp.float32),
            jax.ShapeDtypeStruct((B, 1, 1), jnp.float32),
            jax.ShapeDtypeStruct((B, 1, 1), jnp.float32),
        ],
        compiler_params=pltpu.CompilerParams(
            dimension_semantics=("arbitrary",)),
    )(predicted_scores, labels3, predicted_locs, gt_locs)

    out = pl.pallas_call(
        _stage2_body,
        in_specs=[
            pl.BlockSpec((B, P), lambda: (0, 0)),
            pl.BlockSpec((B, 1, 1), lambda: (0, 0, 0)),
            pl.BlockSpec((B, 1, 1), lambda: (0, 0, 0)),
            pl.BlockSpec((B, 1, 1), lambda: (0, 0, 0)),
        ],
        out_specs=pl.BlockSpec((1, 1), lambda: (0, 0)),
        out_shape=jax.ShapeDtypeStruct((1, 1), jnp.float32),
    )(ce_neg.reshape(B, P), npos, locsum, cepos)

    return jnp.reshape(out, ())


# speedup vs baseline: 3.3327x; 2.7250x over previous
import jax
import jax.numpy as jnp
from jax.experimental import pallas as pl
from jax.experimental.pallas import tpu as pltpu

_C = 81
_NEG_POS_RATIO = 3.0


def _stage1_body(scores_ref, labels_ref, lab4_ref, plocs_ref, glocs_ref,
                 ce_ref, npos_ref, locsum_ref, cepos_ref):
    xt = scores_ref[...].T
    labt = labels_ref[...]
    lab4 = lab4_ref[...]

    mx = jnp.max(xt, axis=0, keepdims=True)
    e = jnp.exp(xt - mx)
    se = jnp.sum(e, axis=0, keepdims=True)
    lse = jnp.log(se) + mx
    cls = jax.lax.broadcasted_iota(jnp.int32, xt.shape, 0)
    xlab = jnp.sum(jnp.where(cls == labt, xt, 0.0), axis=0, keepdims=True)
    ce = lse - xlab
    pos = labt > 0

    d = plocs_ref[...] - glocs_ref[...]
    ad = jnp.abs(d)
    sl1 = jnp.where(ad < 1.0, 0.5 * d * d, ad - 0.5)
    sl1m = jnp.where(lab4 > 0, sl1, 0.0)

    ce_ref[...] = jnp.where(pos, 0.0, ce)
    npos_ref[...] = jnp.sum(jnp.where(pos, 1.0, 0.0)).reshape(1, 1)
    locsum_ref[...] = jnp.sum(sl1m).reshape(1, 1)
    cepos_ref[...] = jnp.sum(jnp.where(pos, ce, 0.0)).reshape(1, 1)


def _stage2_body(ce_ref, npos_ref, locsum_ref, cepos_ref, out_ref):
    ce = jnp.maximum(ce_ref[...], 0.0)
    b, p = ce.shape
    npos = npos_ref[...][:, :, 0]
    negcnt = p - npos
    k = jnp.minimum(_NEG_POS_RATIO * npos, negcnt)

    loc_sum = jnp.sum(locsum_ref[...])
    ce_pos = jnp.sum(cepos_ref[...])
    total_pos = jnp.maximum(jnp.sum(npos), 1.0)

    rowsum = jnp.sum(ce, axis=1, keepdims=True)
    fast = jnp.all((k >= negcnt) | (k <= 0.0))

    @pl.when(fast)
    def _():
        conf = jnp.sum(jnp.where(k > 0.0, rowsum, 0.0))
        out_ref[...] = jnp.reshape((loc_sum + ce_pos + conf) / total_pos, (1, 1))

    @pl.when(jnp.logical_not(fast))
    def _():
        u = jax.lax.bitcast_convert_type(ce, jnp.int32)
        lo0 = jnp.zeros((b, 1), jnp.int32)
        hi0 = jnp.full((b, 1), jnp.int32(0x7F800000))

        def step(_, carry):
            lo, hi = carry
            mid = lo + jax.lax.shift_right_logical(hi - lo, 1)
            cnt = jnp.sum(jnp.where(u >= mid, 1.0, 0.0), axis=1, keepdims=True)
            go = cnt >= k
            return jnp.where(go, mid, lo), jnp.where(go, hi, mid)

        lo, _ = jax.lax.fori_loop(0, 31, step, (lo0, hi0))
        t = jax.lax.bitcast_convert_type(lo, jnp.float32)
        gt = ce > t
        cnt_gt = jnp.sum(jnp.where(gt, 1.0, 0.0), axis=1, keepdims=True)
        sum_gt = jnp.sum(jnp.where(gt, ce, 0.0), axis=1, keepdims=True)
        topk = sum_gt + (k - cnt_gt) * t
        conf = jnp.sum(jnp.where(k > 0.0, topk, 0.0))
        out_ref[...] = jnp.reshape((loc_sum + ce_pos + conf) / total_pos, (1, 1))


def kernel(predicted_locs, predicted_scores, gt_locs, gt_labels):
    B, P, C = predicted_scores.shape
    labels3 = gt_labels.astype(jnp.int32).reshape(B, 1, P)
    lab4 = jnp.broadcast_to(gt_labels.astype(jnp.int32)[:, :, None],
                            (B, P, 4)).reshape(B, 1, 4 * P)
    plocs_f = predicted_locs.reshape(B, 1, 4 * P)
    glocs_f = gt_locs.reshape(B, 1, 4 * P)

    ce_neg, npos, locsum, cepos = pl.pallas_call(
        _stage1_body,
        grid=(B,),
        in_specs=[
            pl.BlockSpec((None, P, C), lambda b: (b, 0, 0)),
            pl.BlockSpec((None, 1, P), lambda b: (b, 0, 0)),
            pl.BlockSpec((None, 1, 4 * P), lambda b: (b, 0, 0)),
            pl.BlockSpec((None, 1, 4 * P), lambda b: (b, 0, 0)),
            pl.BlockSpec((None, 1, 4 * P), lambda b: (b, 0, 0)),
        ],
        out_specs=[
            pl.BlockSpec((None, 1, P), lambda b: (b, 0, 0)),
            pl.BlockSpec((None, 1, 1), lambda b: (b, 0, 0)),
            pl.BlockSpec((None, 1, 1), lambda b: (b, 0, 0)),
            pl.BlockSpec((None, 1, 1), lambda b: (b, 0, 0)),
        ],
        out_shape=[
            jax.ShapeDtypeStruct((B, 1, P), jnp.float32),
            jax.ShapeDtypeStruct((B, 1, 1), jnp.float32),
            jax.ShapeDtypeStruct((B, 1, 1), jnp.float32),
            jax.ShapeDtypeStruct((B, 1, 1), jnp.float32),
        ],
        compiler_params=pltpu.CompilerParams(
            dimension_semantics=("arbitrary",)),
    )(predicted_scores, labels3, lab4, plocs_f, glocs_f)

    out = pl.pallas_call(
        _stage2_body,
        in_specs=[
            pl.BlockSpec((B, P), lambda: (0, 0)),
            pl.BlockSpec((B, 1, 1), lambda: (0, 0, 0)),
            pl.BlockSpec((B, 1, 1), lambda: (0, 0, 0)),
            pl.BlockSpec((B, 1, 1), lambda: (0, 0, 0)),
        ],
        out_specs=pl.BlockSpec((1, 1), lambda: (0, 0)),
        out_shape=jax.ShapeDtypeStruct((1, 1), jnp.float32),
    )(ce_neg.reshape(B, P), npos, locsum, cepos)

    return jnp.reshape(out, ())


# speedup vs baseline: 3.8346x; 1.1506x over previous
import jax
import jax.numpy as jnp
from jax.experimental import pallas as pl
from jax.experimental.pallas import tpu as pltpu

_C = 81
_NEG_POS_RATIO = 3.0


def _stage1_body(scores_ref, labels_ref, ld_ref,
                 ce_ref, npos_ref, locsum_ref, cepos_ref):
    xt = scores_ref[...].T
    labt = labels_ref[...]

    mx = jnp.max(xt, axis=0, keepdims=True)
    e = jnp.exp(xt - mx)
    se = jnp.sum(e, axis=0, keepdims=True)
    lse = jnp.log(se) + mx
    cls = jax.lax.broadcasted_iota(jnp.int32, xt.shape, 0)
    xlab = jnp.sum(jnp.where(cls == labt, xt, 0.0), axis=0, keepdims=True)
    ce = lse - xlab
    pos = labt > 0

    d = ld_ref[...]
    ad = jnp.abs(d)
    sl1 = jnp.where(ad < 1.0, 0.5 * d * d, ad - 0.5)

    ce_ref[...] = jnp.where(pos, 0.0, ce)
    npos_ref[...] = jnp.sum(jnp.where(pos, 1.0, 0.0)).reshape(1, 1)
    locsum_ref[...] = jnp.sum(sl1).reshape(1, 1)
    cepos_ref[...] = jnp.sum(jnp.where(pos, ce, 0.0)).reshape(1, 1)


def _stage2_body(ce_ref, npos_ref, locsum_ref, cepos_ref, out_ref):
    ce = jnp.maximum(ce_ref[...], 0.0)
    b, p = ce.shape
    npos = npos_ref[...][:, :, 0]
    negcnt = p - npos
    k = jnp.minimum(_NEG_POS_RATIO * npos, negcnt)

    loc_sum = jnp.sum(locsum_ref[...])
    ce_pos = jnp.sum(cepos_ref[...])
    total_pos = jnp.maximum(jnp.sum(npos), 1.0)

    rowsum = jnp.sum(ce, axis=1, keepdims=True)
    fast = jnp.all((k >= negcnt) | (k <= 0.0))

    @pl.when(fast)
    def _():
        conf = jnp.sum(jnp.where(k > 0.0, rowsum, 0.0))
        out_ref[...] = jnp.reshape((loc_sum + ce_pos + conf) / total_pos, (1, 1))

    @pl.when(jnp.logical_not(fast))
    def _():
        u = jax.lax.bitcast_convert_type(ce, jnp.int32)
        lo0 = jnp.zeros((b, 1), jnp.int32)
        hi0 = jnp.full((b, 1), jnp.int32(0x7F800000))

        def step(_, carry):
            lo, hi = carry
            mid = lo + jax.lax.shift_right_logical(hi - lo, 1)
            cnt = jnp.sum(jnp.where(u >= mid, 1.0, 0.0), axis=1, keepdims=True)
            go = cnt >= k
            return jnp.where(go, mid, lo), jnp.where(go, hi, mid)

        lo, _ = jax.lax.fori_loop(0, 31, step, (lo0, hi0))
        t = jax.lax.bitcast_convert_type(lo, jnp.float32)
        gt = ce > t
        cnt_gt = jnp.sum(jnp.where(gt, 1.0, 0.0), axis=1, keepdims=True)
        sum_gt = jnp.sum(jnp.where(gt, ce, 0.0), axis=1, keepdims=True)
        topk = sum_gt + (k - cnt_gt) * t
        conf = jnp.sum(jnp.where(k > 0.0, topk, 0.0))
        out_ref[...] = jnp.reshape((loc_sum + ce_pos + conf) / total_pos, (1, 1))


def kernel(predicted_locs, predicted_scores, gt_locs, gt_labels):
    B, P, C = predicted_scores.shape
    labels3 = gt_labels.astype(jnp.int32).reshape(B, 1, P)
    ld = jnp.where(gt_labels[:, :, None] > 0,
                   predicted_locs - gt_locs, 0.0).reshape(B, 1, 4 * P)

    ce_neg, npos, locsum, cepos = pl.pallas_call(
        _stage1_body,
        grid=(B,),
        in_specs=[
            pl.BlockSpec((None, P, C), lambda b: (b, 0, 0)),
            pl.BlockSpec((None, 1, P), lambda b: (b, 0, 0)),
            pl.BlockSpec((None, 1, 4 * P), lambda b: (b, 0, 0)),
        ],
        out_specs=[
            pl.BlockSpec((None, 1, P), lambda b: (b, 0, 0)),
            pl.BlockSpec((None, 1, 1), lambda b: (b, 0, 0)),
            pl.BlockSpec((None, 1, 1), lambda b: (b, 0, 0)),
            pl.BlockSpec((None, 1, 1), lambda b: (b, 0, 0)),
        ],
        out_shape=[
            jax.ShapeDtypeStruct((B, 1, P), jnp.float32),
            jax.ShapeDtypeStruct((B, 1, 1), jnp.float32),
            jax.ShapeDtypeStruct((B, 1, 1), jnp.float32),
            jax.ShapeDtypeStruct((B, 1, 1), jnp.float32),
        ],
        compiler_params=pltpu.CompilerParams(
            dimension_semantics=("arbitrary",)),
    )(predicted_scores, labels3, ld)

    out = pl.pallas_call(
        _stage2_body,
        in_specs=[
            pl.BlockSpec((B, P), lambda: (0, 0)),
            pl.BlockSpec((B, 1, 1), lambda: (0, 0, 0)),
            pl.BlockSpec((B, 1, 1), lambda: (0, 0, 0)),
            pl.BlockSpec((B, 1, 1), lambda: (0, 0, 0)),
        ],
        out_specs=pl.BlockSpec((1, 1), lambda: (0, 0)),
        out_shape=jax.ShapeDtypeStruct((1, 1), jnp.float32),
    )(ce_neg.reshape(B, P), npos, locsum, cepos)

    return jnp.reshape(out, ())


# speedup vs baseline: 3.8996x; 1.0169x over previous
import jax
import jax.numpy as jnp
from jax.experimental import pallas as pl
from jax.experimental.pallas import tpu as pltpu

_NEG_POS_RATIO = 3.0


def _body(scores_ref, labels_ref, ld_ref, out_ref, ce_buf, aux_buf):
    b = pl.program_id(0)
    nb = pl.num_programs(0)

    xt = scores_ref[...].T
    labt = labels_ref[...]
    p = xt.shape[1]

    mx = jnp.max(xt, axis=0, keepdims=True)
    e = jnp.exp(xt - mx)
    se = jnp.sum(e, axis=0, keepdims=True)
    lse = jnp.log(se) + mx
    cls = jax.lax.broadcasted_iota(jnp.int32, xt.shape, 0)
    xlab = jnp.sum(jnp.where(cls == labt, xt, 0.0), axis=0, keepdims=True)
    ce = lse - xlab
    pos = labt > 0

    d = ld_ref[...]
    ad = jnp.abs(d)
    sl1 = jnp.where(ad < 1.0, 0.5 * d * d, ad - 0.5)

    ce_buf[pl.ds(b, 1), :] = jnp.maximum(jnp.where(pos, 0.0, ce), 0.0)

    npos = jnp.sum(jnp.where(pos, 1.0, 0.0))
    locsum = jnp.sum(sl1)
    cepos = jnp.sum(jnp.where(pos, ce, 0.0))
    li = jax.lax.broadcasted_iota(jnp.int32, (1, 128), 1)
    aux = jnp.where(li == 0, npos,
                    jnp.where(li == 1, locsum,
                              jnp.where(li == 2, cepos, 0.0)))
    aux_buf[pl.ds(b, 1), :] = aux

    @pl.when(b == nb - 1)
    def _finalize():
        ce_all = ce_buf[...]
        aux_all = aux_buf[...]
        nrows = ce_all.shape[0]
        npos_c = aux_all[:, 0:1]
        negcnt = p - npos_c
        k = jnp.minimum(_NEG_POS_RATIO * npos_c, negcnt)

        loc_sum = jnp.sum(aux_all[:, 1:2])
        ce_pos = jnp.sum(aux_all[:, 2:3])
        total_pos = jnp.maximum(jnp.sum(npos_c), 1.0)

        rowsum = jnp.sum(ce_all, axis=1, keepdims=True)
        fast = jnp.all((k >= negcnt) | (k <= 0.0))

        @pl.when(fast)
        def _():
            conf = jnp.sum(jnp.where(k > 0.0, rowsum, 0.0))
            out_ref[...] = jnp.reshape(
                (loc_sum + ce_pos + conf) / total_pos, (1, 1))

        @pl.when(jnp.logical_not(fast))
        def _():
            u = jax.lax.bitcast_convert_type(ce_all, jnp.int32)
            lo0 = jnp.zeros((nrows, 1), jnp.int32)
            hi0 = jnp.full((nrows, 1), jnp.int32(0x7F800000))

            def step(_, carry):
                lo, hi = carry
                mid = lo + jax.lax.shift_right_logical(hi - lo, 1)
                cnt = jnp.sum(jnp.where(u >= mid, 1.0, 0.0),
                              axis=1, keepdims=True)
                go = cnt >= k
                return jnp.where(go, mid, lo), jnp.where(go, hi, mid)

            lo, _ = jax.lax.fori_loop(0, 31, step, (lo0, hi0))
            t = jax.lax.bitcast_convert_type(lo, jnp.float32)
            gt = ce_all > t
            cnt_gt = jnp.sum(jnp.where(gt, 1.0, 0.0), axis=1, keepdims=True)
            sum_gt = jnp.sum(jnp.where(gt, ce_all, 0.0), axis=1, keepdims=True)
            topk = sum_gt + (k - cnt_gt) * t
            conf = jnp.sum(jnp.where(k > 0.0, topk, 0.0))
            out_ref[...] = jnp.reshape(
                (loc_sum + ce_pos + conf) / total_pos, (1, 1))


def kernel(predicted_locs, predicted_scores, gt_locs, gt_labels):
    B, P, C = predicted_scores.shape
    labels3 = gt_labels.astype(jnp.int32).reshape(B, 1, P)
    ld = jnp.where(gt_labels[:, :, None] > 0,
                   predicted_locs - gt_locs, 0.0).reshape(B, 1, 4 * P)

    out = pl.pallas_call(
        _body,
        grid=(B,),
        in_specs=[
            pl.BlockSpec((None, P, C), lambda b: (b, 0, 0)),
            pl.BlockSpec((None, 1, P), lambda b: (b, 0, 0)),
            pl.BlockSpec((None, 1, 4 * P), lambda b: (b, 0, 0)),
        ],
        out_specs=pl.BlockSpec((1, 1), lambda b: (0, 0)),
        out_shape=jax.ShapeDtypeStruct((1, 1), jnp.float32),
        scratch_shapes=[
            pltpu.VMEM((B, P), jnp.float32),
            pltpu.VMEM((B, 128), jnp.float32),
        ],
        compiler_params=pltpu.CompilerParams(
            dimension_semantics=("arbitrary",)),
    )(predicted_scores, labels3, ld)

    return jnp.reshape(out, ())
